# baseline (speedup 1.0000x reference)
import functools

import jax
import jax.numpy as jnp
from jax import lax
from jax.experimental import pallas as pl
from jax.experimental.pallas import tpu as pltpu
from jax.experimental.pallas import tpu_sc as plsc

INV_QUAN = 50.0
NHEAD = 16
LANES = 16
NC, NS = 2, 16
NW = NC * NS


@functools.lru_cache(maxsize=None)
def _rpe_sc_kernel(max_len, n_points):
    ppw = n_points // NW
    SLAB = 64
    n_slabs = ppw // SLAB
    g_per_slab = SLAB // LANES

    mesh = plsc.VectorSubcoreMesh(core_axis_name="c", subcore_axis_name="s")

    @functools.partial(
        pl.kernel,
        mesh=mesh,
        out_type=jax.ShapeDtypeStruct((NW, ppw, NHEAD), jnp.float32),
        compiler_params=pltpu.CompilerParams(needs_layout_passes=False),
        scratch_types=[
            pltpu.VMEM((max_len * NHEAD,), jnp.float32),
            pltpu.VMEM((ppw,), jnp.float32),
            pltpu.VMEM((ppw,), jnp.float32),
            pltpu.VMEM((ppw,), jnp.float32),
            pltpu.VMEM((SLAB, NHEAD), jnp.float32),
            pltpu.VMEM((SLAB, NHEAD), jnp.float32),
            pltpu.VMEM((SLAB, NHEAD), jnp.float32),
            pltpu.VMEM((SLAB, NHEAD), jnp.float32),
            pltpu.SemaphoreType.DMA,
            pltpu.SemaphoreType.DMA,
            pltpu.SemaphoreType.DMA,
            pltpu.SemaphoreType.DMA,
        ],
    )
    def k(x_hbm, y_hbm, z_hbm, table_hbm, out_hbm,
          tbl_v, x_v, y_v, z_v, ob0_v, ob1_v, ob2_v, ob3_v,
          sem0, sem1, sem2, sem3):
        wid = lax.axis_index("s") * NC + lax.axis_index("c")
        base = wid * ppw
        tbl_cp = pltpu.async_copy(table_hbm, tbl_v, sem0)
        pltpu.sync_copy(x_hbm.at[pl.ds(base, ppw)], x_v)
        pltpu.sync_copy(y_hbm.at[pl.ds(base, ppw)], y_v)
        pltpu.sync_copy(z_hbm.at[pl.ds(base, ppw)], z_v)
        tbl_cp.wait()

        lanes = lax.iota(jnp.int32, LANES)
        zero = lanes * 0

        def group(out_v, slab, g):
            off = slab * SLAB + g * LANES
            x = x_v[pl.ds(off, LANES)]
            y = y_v[pl.ds(off, LANES)]
            z = z_v[pl.ds(off, LANES)]
            s = jnp.maximum(x * x + y * y + z * z, 1e-30)
            bits = lax.bitcast_convert_type(s, jnp.int32)
            r = lax.bitcast_convert_type(0x5F3759DF - (bits >> 1), jnp.float32)
            hs = 0.5 * s
            r = r * (1.5 - hs * r * r)
            r = r * (1.5 - hs * r * r)
            r = r * (1.5 - hs * r * r)
            d = s * r * INV_QUAN
            i1 = d.astype(jnp.int32)
            i2 = i1 + 1
            w1 = i2.astype(jnp.float32) - d
            w2 = d - i1.astype(jnp.float32)
            b1 = jnp.minimum(i1, max_len - 1) * NHEAD
            b2 = jnp.minimum(i2, max_len - 1) * NHEAD
            orow = g * LANES + lanes
            for c in range(NHEAD):
                e1 = plsc.load_gather(tbl_v, [b1 + c])
                e2 = plsc.load_gather(tbl_v, [b2 + c])
                plsc.store_scatter(out_v, [orow, zero + c], e1 * w1 + e2 * w2)

        bufs = (ob0_v, ob1_v, ob2_v, ob3_v)
        sems = (sem0, sem1, sem2, sem3)

        def slab_pair(j, _):
            for b in range(4):
                slab = j * 4 + b

                @pl.when(j > 0)
                def _():
                    pltpu.make_async_copy(
                        bufs[b], out_hbm.at[wid, pl.ds(0, SLAB), :], sems[b]
                    ).wait()

                plsc.parallel_loop(0, g_per_slab, unroll=1)(
                    functools.partial(group, bufs[b], slab)
                )
                pltpu.make_async_copy(
                    bufs[b], out_hbm.at[wid, pl.ds(slab * SLAB, SLAB), :], sems[b]
                ).start()
            return 0

        lax.fori_loop(0, n_slabs // 4, slab_pair, 0)
        for b in range(4):
            pltpu.make_async_copy(
                bufs[b], out_hbm.at[wid, pl.ds(0, SLAB), :], sems[b]
            ).wait()

    return k


def kernel(batch_rel_coords, pos_embed):
    nb, np_, _ = batch_rel_coords.shape
    n = nb * np_
    max_len = pos_embed.shape[0]
    out = _rpe_sc_kernel(max_len, n)(
        batch_rel_coords[:, :, 0].reshape(-1),
        batch_rel_coords[:, :, 1].reshape(-1),
        batch_rel_coords[:, :, 2].reshape(-1),
        pos_embed.reshape(-1),
    )
    return out.reshape(nb, np_, NHEAD)


# speedup vs baseline: 1.1957x; 1.1957x over previous
import functools

import jax
import jax.numpy as jnp
from jax import lax
from jax.experimental import pallas as pl
from jax.experimental.pallas import tpu as pltpu
from jax.experimental.pallas import tpu_sc as plsc

INV_QUAN = 50.0
NHEAD = 16
LANES = 16
NC, NS = 2, 16
NW = NC * NS


@functools.lru_cache(maxsize=None)
def _rpe_sc_kernel(max_len, n_points):
    ppw = n_points // NW
    SLAB = 128
    n_slabs = ppw // SLAB
    g_per_slab = SLAB // LANES

    mesh = plsc.VectorSubcoreMesh(core_axis_name="c", subcore_axis_name="s")

    @functools.partial(
        pl.kernel,
        mesh=mesh,
        out_type=jax.ShapeDtypeStruct((NW, ppw, NHEAD), jnp.float32),
        compiler_params=pltpu.CompilerParams(needs_layout_passes=False),
        scratch_types=[
            pltpu.VMEM((max_len * NHEAD,), jnp.float32),
            pltpu.VMEM((ppw,), jnp.float32),
            pltpu.VMEM((ppw,), jnp.float32),
            pltpu.VMEM((ppw,), jnp.float32),
            pltpu.VMEM((SLAB, NHEAD), jnp.float32),
            pltpu.VMEM((SLAB, NHEAD), jnp.float32),
            pltpu.SemaphoreType.DMA,
            pltpu.SemaphoreType.DMA,
        ],
    )
    def k(x_hbm, y_hbm, z_hbm, table_hbm, out_hbm,
          tbl_v, x_v, y_v, z_v, ob0_v, ob1_v, sem0, sem1):
        wid = lax.axis_index("s") * NC + lax.axis_index("c")
        base = wid * ppw
        tbl_cp = pltpu.async_copy(table_hbm, tbl_v, sem0)
        pltpu.sync_copy(x_hbm.at[pl.ds(base, ppw)], x_v)
        pltpu.sync_copy(y_hbm.at[pl.ds(base, ppw)], y_v)
        pltpu.sync_copy(z_hbm.at[pl.ds(base, ppw)], z_v)

        lanes = lax.iota(jnp.int32, LANES)
        zero = lanes * 0
        bmax = (max_len - 1) * NHEAD

        @plsc.parallel_loop(0, ppw // LANES, unroll=1)
        def _(g):
            off = g * LANES
            x = x_v[pl.ds(off, LANES)]
            y = y_v[pl.ds(off, LANES)]
            z = z_v[pl.ds(off, LANES)]
            s = jnp.maximum(x * x + y * y + z * z, 1e-30)
            bits = lax.bitcast_convert_type(s, jnp.int32)
            r = lax.bitcast_convert_type(0x5F3759DF - (bits >> 1), jnp.float32)
            hs = 0.5 * s
            r = r * (1.5 - hs * r * r)
            r = r * (1.5 - hs * r * r)
            r = r * (1.5 - hs * r * r)
            d = s * r * INV_QUAN
            i1 = d.astype(jnp.int32)
            w1 = (i1 + 1).astype(jnp.float32) - d
            b1 = jnp.minimum(i1, max_len - 1) * NHEAD
            x_v[pl.ds(off, LANES)] = lax.bitcast_convert_type(b1, jnp.float32)
            y_v[pl.ds(off, LANES)] = w1

        tbl_cp.wait()

        def group(out_v, slab, g):
            off = slab * SLAB + g * LANES
            b1 = lax.bitcast_convert_type(x_v[pl.ds(off, LANES)], jnp.int32)
            w1 = y_v[pl.ds(off, LANES)]
            b2 = jnp.minimum(b1 + NHEAD, bmax)
            w2 = 1.0 - w1
            orow = g * LANES + lanes
            for c in range(NHEAD):
                e1 = plsc.load_gather(tbl_v, [b1 + c])
                e2 = plsc.load_gather(tbl_v, [b2 + c])
                plsc.store_scatter(out_v, [orow, zero + c], e1 * w1 + e2 * w2)

        bufs = (ob0_v, ob1_v)
        sems = (sem0, sem1)

        def slab_pair(j, _):
            for b in range(2):
                slab = j * 2 + b

                @pl.when(j > 0)
                def _():
                    pltpu.make_async_copy(
                        bufs[b], out_hbm.at[wid, pl.ds(0, SLAB), :], sems[b]
                    ).wait()

                plsc.parallel_loop(0, g_per_slab, unroll=1)(
                    functools.partial(group, bufs[b], slab)
                )
                pltpu.make_async_copy(
                    bufs[b], out_hbm.at[wid, pl.ds(slab * SLAB, SLAB), :], sems[b]
                ).start()
            return 0

        lax.fori_loop(0, n_slabs // 2, slab_pair, 0)
        for b in range(2):
            pltpu.make_async_copy(
                bufs[b], out_hbm.at[wid, pl.ds(0, SLAB), :], sems[b]
            ).wait()

    return k


def kernel(batch_rel_coords, pos_embed):
    nb, np_, _ = batch_rel_coords.shape
    n = nb * np_
    max_len = pos_embed.shape[0]
    out = _rpe_sc_kernel(max_len, n)(
        batch_rel_coords[:, :, 0].reshape(-1),
        batch_rel_coords[:, :, 1].reshape(-1),
        batch_rel_coords[:, :, 2].reshape(-1),
        pos_embed.reshape(-1),
    )
    return out.reshape(nb, np_, NHEAD)


# speedup vs baseline: 1.2005x; 1.0040x over previous
import functools

import jax
import jax.numpy as jnp
from jax import lax
from jax.experimental import pallas as pl
from jax.experimental.pallas import tpu as pltpu
from jax.experimental.pallas import tpu_sc as plsc

INV_QUAN = 50.0
NHEAD = 16
LANES = 16
NC, NS = 2, 16
NW = NC * NS


@functools.lru_cache(maxsize=None)
def _rpe_sc_kernel(max_len, n_points):
    ppw = n_points // NW
    SLAB = 128
    n_slabs = ppw // SLAB
    g_per_slab = SLAB // LANES

    mesh = plsc.VectorSubcoreMesh(core_axis_name="c", subcore_axis_name="s")

    @functools.partial(
        pl.kernel,
        mesh=mesh,
        out_type=jax.ShapeDtypeStruct((NW, ppw, NHEAD), jnp.float32),
        compiler_params=pltpu.CompilerParams(needs_layout_passes=False),
        scratch_types=[
            pltpu.VMEM((max_len * NHEAD,), jnp.float32),
            pltpu.VMEM((ppw,), jnp.float32),
            pltpu.VMEM((ppw,), jnp.float32),
            pltpu.VMEM((ppw,), jnp.float32),
            pltpu.VMEM((SLAB, NHEAD), jnp.float32),
            pltpu.VMEM((SLAB, NHEAD), jnp.float32),
            pltpu.SemaphoreType.DMA,
            pltpu.SemaphoreType.DMA,
        ],
    )
    def k(x_hbm, y_hbm, z_hbm, table_hbm, out_hbm,
          tbl_v, x_v, y_v, z_v, ob0_v, ob1_v, sem0, sem1):
        wid = lax.axis_index("s") * NC + lax.axis_index("c")
        base = wid * ppw
        tbl_cp = pltpu.async_copy(table_hbm, tbl_v, sem0)
        pltpu.sync_copy(x_hbm.at[pl.ds(base, ppw)], x_v)
        pltpu.sync_copy(y_hbm.at[pl.ds(base, ppw)], y_v)
        pltpu.sync_copy(z_hbm.at[pl.ds(base, ppw)], z_v)

        lanes = lax.iota(jnp.int32, LANES)
        zero = lanes * 0
        bmax = (max_len - 1) * NHEAD

        @plsc.parallel_loop(0, ppw // LANES, unroll=2)
        def _(g):
            off = g * LANES
            x = x_v[pl.ds(off, LANES)]
            y = y_v[pl.ds(off, LANES)]
            z = z_v[pl.ds(off, LANES)]
            s = jnp.maximum(x * x + y * y + z * z, 1e-30)
            bits = lax.bitcast_convert_type(s, jnp.int32)
            r = lax.bitcast_convert_type(0x5F3759DF - (bits >> 1), jnp.float32)
            hs = 0.5 * s
            r = r * (1.5 - hs * r * r)
            r = r * (1.5 - hs * r * r)
            r = r * (1.5 - hs * r * r)
            d = s * r * INV_QUAN
            i1 = d.astype(jnp.int32)
            w1 = (i1 + 1).astype(jnp.float32) - d
            b1 = jnp.minimum(i1, max_len - 1) * NHEAD
            x_v[pl.ds(off, LANES)] = lax.bitcast_convert_type(b1, jnp.float32)
            y_v[pl.ds(off, LANES)] = w1

        tbl_cp.wait()

        def group(out_v, slab, g):
            off = slab * SLAB + g * LANES
            b1 = lax.bitcast_convert_type(x_v[pl.ds(off, LANES)], jnp.int32)
            w1 = y_v[pl.ds(off, LANES)]
            b2 = jnp.minimum(b1 + NHEAD, bmax)
            w2 = 1.0 - w1
            orow = g * LANES + lanes
            for c in range(NHEAD):
                e1 = plsc.load_gather(tbl_v, [b1 + c])
                e2 = plsc.load_gather(tbl_v, [b2 + c])
                plsc.store_scatter(out_v, [orow, zero + c], e1 * w1 + e2 * w2)

        bufs = (ob0_v, ob1_v)
        sems = (sem0, sem1)

        def slab_pair(j, _):
            for b in range(2):
                slab = j * 2 + b

                @pl.when(j > 0)
                def _():
                    pltpu.make_async_copy(
                        bufs[b], out_hbm.at[wid, pl.ds(0, SLAB), :], sems[b]
                    ).wait()

                plsc.parallel_loop(0, g_per_slab, unroll=1)(
                    functools.partial(group, bufs[b], slab)
                )
                pltpu.make_async_copy(
                    bufs[b], out_hbm.at[wid, pl.ds(slab * SLAB, SLAB), :], sems[b]
                ).start()
            return 0

        lax.fori_loop(0, n_slabs // 2, slab_pair, 0)
        for b in range(2):
            pltpu.make_async_copy(
                bufs[b], out_hbm.at[wid, pl.ds(0, SLAB), :], sems[b]
            ).wait()

    return k


def kernel(batch_rel_coords, pos_embed):
    nb, np_, _ = batch_rel_coords.shape
    n = nb * np_
    max_len = pos_embed.shape[0]
    out = _rpe_sc_kernel(max_len, n)(
        batch_rel_coords[:, :, 0].reshape(-1),
        batch_rel_coords[:, :, 1].reshape(-1),
        batch_rel_coords[:, :, 2].reshape(-1),
        pos_embed.reshape(-1),
    )
    return out.reshape(nb, np_, NHEAD)
